# baseline (speedup 1.0000x reference)
import functools

import jax
import jax.numpy as jnp
from jax import lax
from jax.experimental import pallas as pl
from jax.experimental.pallas import tpu as pltpu
from jax.experimental.pallas import tpu_sc as plsc

N_NODES = 10000
N_EDGES = 320000
D_IN = 128
D_HID = 128
D_OUT = 64
N_GRAPHS = 512
N_IDX = 256

NP = 10112
NBLK = N_EDGES // 128
NSEX = 2496 // 6
SEXT = NSEX // 16
SLAB = 6 * SEXT + 4
BBLK = NP // 128
GP = N_GRAPHS + 8
ROWS_PER_TILE = NP // 16
CSLAB = NBLK // 32 + 1

_mesh = functools.partial(
    plsc.VectorSubcoreMesh, core_axis_name="c", subcore_axis_name="s")
_sc_params = pltpu.CompilerParams(use_tc_tiling_on_sc=False)


def _span(total, parts, p):
    base = total // parts
    rem = total % parts
    lo = p * base + jnp.minimum(p, rem)
    hi = lo + base + jnp.where(p < rem, 1, 0)
    return lo, hi


def _make_counts_kernel():
    @functools.partial(
        pl.kernel,
        out_type=(
            jax.ShapeDtypeStruct((2, NP, 16), jnp.float32),
            jax.ShapeDtypeStruct((2, GP, 16), jnp.float32),
        ),
        mesh=_mesh(),
        compiler_params=_sc_params,
        scratch_types=[
            pltpu.VMEM_SHARED((NP, 16), jnp.float32),
            pltpu.VMEM_SHARED((GP, 16), jnp.float32),
            pltpu.VMEM((CSLAB, 1, 128), jnp.int32),
            pltpu.VMEM((1, 128), jnp.int32),
            pltpu.VMEM((128, 16), jnp.float32),
            [pltpu.SemaphoreType.DMA] * 4,
        ],
    )
    def counts(dst_hbm, batch_hbm, ones_hbm, zeros1_hbm,
               deg_out, cnt_out, deg_sh, cnt_sh, islab, idx_v, ones_v, sems):
        c = lax.axis_index("c")
        s = lax.axis_index("s")
        w = c * 16 + s

        blo, bhi = _span(NBLK, 32, w)
        nb = bhi - blo
        pltpu.sync_copy(dst_hbm.at[pl.ds(blo, CSLAB)], islab)
        pltpu.sync_copy(ones_hbm, ones_v)
        pltpu.sync_copy(zeros1_hbm.at[pl.ds(0, ROWS_PER_TILE)],
                        deg_sh.at[pl.ds(s * ROWS_PER_TILE, ROWS_PER_TILE)])

        @pl.when(s == 0)
        def _zc():
            pltpu.sync_copy(zeros1_hbm.at[pl.ds(0, GP)], cnt_sh)

        plsc.subcore_barrier()

        def fire(i, k):
            pltpu.async_copy(ones_v, deg_sh.at[islab.at[i].at[0]], sems[k],
                             add=True)

        def drain(k):
            pltpu.make_async_copy(ones_v, deg_sh.at[islab.at[0].at[0]],
                                  sems[k]).wait()

        for k in range(4):
            fire(k, k)

        def body4(q, carry):
            i0 = 4 + q * 4
            for k in range(4):
                i = i0 + k

                @pl.when(i < nb)
                def _w(i=i, k=k):
                    drain(k)
                    fire(i, k)
            return carry

        lax.fori_loop(0, (nb + 3) // 4, body4, 0)

        for k in range(4):
            drain(k)

        lo2, hi2 = _span(BBLK, 32, w)

        def cnt_body(j, carry):
            pltpu.sync_copy(batch_hbm.at[j], idx_v)
            pltpu.sync_copy(ones_v, cnt_sh.at[idx_v.at[0]], add=True)
            return carry

        lax.fori_loop(lo2, hi2, cnt_body, 0)

        plsc.subcore_barrier()

        pltpu.sync_copy(deg_sh.at[pl.ds(s * ROWS_PER_TILE, ROWS_PER_TILE)],
                        deg_out.at[c].at[pl.ds(s * ROWS_PER_TILE,
                                               ROWS_PER_TILE)])

        @pl.when(s == 0)
        def _co():
            pltpu.sync_copy(cnt_sh, cnt_out.at[c])

    return counts


def _make_edge_kernel():
    @functools.partial(
        pl.kernel,
        out_type=jax.ShapeDtypeStruct((2, NP, 64), jnp.float32),
        mesh=_mesh(),
        compiler_params=_sc_params,
        scratch_types=[
            pltpu.VMEM_SHARED((NP, 64), jnp.float32),
            pltpu.VMEM((SLAB, 1, 128), jnp.int32),
            pltpu.VMEM((SLAB, 1, 128), jnp.int32),
            pltpu.VMEM((6, 128, 64), jnp.float32),
            pltpu.SemaphoreType.DMA,
            pltpu.SemaphoreType.DMA,
            [pltpu.SemaphoreType.DMA] * 6,
            [pltpu.SemaphoreType.DMA] * 6,
        ],
    )
    def edge_pass(src_hbm, dst_hbm, table_hbm, zeros_hbm, acc_out,
                  acc_sh, sslab, dslab, rows, isem0, isem1, gsems, ssems):
        c = lax.axis_index("c")
        s = lax.axis_index("s")
        tbl = table_hbm.at[c]

        olo = s * SEXT
        ohi = olo + SEXT

        pltpu.async_copy(src_hbm.at[pl.ds(6 * olo, SLAB)], sslab, isem0)
        pltpu.async_copy(dst_hbm.at[pl.ds(6 * olo, SLAB)], dslab, isem1)
        pltpu.sync_copy(zeros_hbm,
                        acc_sh.at[pl.ds(s * ROWS_PER_TILE, ROWS_PER_TILE)])
        pltpu.make_async_copy(src_hbm.at[pl.ds(6 * olo, SLAB)], sslab,
                              isem0).wait()
        pltpu.make_async_copy(dst_hbm.at[pl.ds(6 * olo, SLAB)], dslab,
                              isem1).wait()
        plsc.subcore_barrier()

        def gather(l, k):
            pltpu.async_copy(tbl.at[sslab.at[l].at[0]], rows.at[k], gsems[k])

        def gather_wait(k):
            pltpu.make_async_copy(tbl.at[sslab.at[0].at[0]], rows.at[k],
                                  gsems[k]).wait()

        def scat(l, k):
            pltpu.async_copy(rows.at[k], acc_sh.at[dslab.at[l].at[0]],
                             ssems[k], add=True)

        def scat_wait(k):
            pltpu.make_async_copy(rows.at[k], acc_sh.at[dslab.at[0].at[0]],
                                  ssems[k]).wait()

        for k in range(6):
            gather(k, k)

        def body(o, carry):
            l0 = 6 * (o - olo)
            for k in range(6):
                gather_wait(k)
                scat(l0 + k, k)
            for k in range(6):
                scat_wait(k)

                @pl.when(o + 1 < ohi)
                def _pf(k=k, l0=l0):
                    gather(l0 + 6 + k, k)
            return carry

        lax.fori_loop(olo, ohi, body, 0)

        @pl.when(s == 15)
        def _tail():
            for k in range(4):
                gather(156 + k, k)
            for k in range(4):
                gather_wait(k)
                scat(156 + k, k)
            for k in range(4):
                scat_wait(k)

        plsc.subcore_barrier()

        pltpu.sync_copy(acc_sh.at[pl.ds(s * ROWS_PER_TILE, ROWS_PER_TILE)],
                        acc_out.at[c].at[pl.ds(s * ROWS_PER_TILE,
                                               ROWS_PER_TILE)])

    return edge_pass


def _make_pool_kernel():
    @functools.partial(
        pl.kernel,
        out_type=jax.ShapeDtypeStruct((2, GP, 64), jnp.float32),
        mesh=_mesh(),
        compiler_params=_sc_params,
        scratch_types=[
            pltpu.VMEM_SHARED((GP, 64), jnp.float32),
            pltpu.VMEM((1, 128), jnp.int32),
            pltpu.VMEM((128, 64), jnp.float32),
        ],
    )
    def pool_pass(batch_hbm, table_hbm, zeros_hbm,
                  pool_out, acc_sh, dstv, rows_v):
        c = lax.axis_index("c")
        s = lax.axis_index("s")

        @pl.when(s == 0)
        def _z():
            pltpu.sync_copy(zeros_hbm.at[pl.ds(0, GP)], acc_sh)

        plsc.subcore_barrier()

        lo, hi = _span(BBLK, 16, s)

        def body(j, carry):
            pltpu.sync_copy(batch_hbm.at[j], dstv)
            pltpu.sync_copy(table_hbm.at[c].at[pl.ds(j * 128, 128)], rows_v)
            pltpu.sync_copy(rows_v, acc_sh.at[dstv.at[0]], add=True)
            return carry

        lax.fori_loop(lo, hi, body, 0)

        plsc.subcore_barrier()

        @pl.when(s == 0)
        def _co():
            pltpu.sync_copy(acc_sh, pool_out.at[c])

    return pool_pass


RB = NP // 8


def _prep_kernel(x_ref, w_ref, deg_ref, zs_ref, dinv_ref):
    deg = deg_ref[0][:, 0:1] + deg_ref[1][:, 0:1] + 1.0
    dv = lax.rsqrt(deg)
    z = jnp.dot(x_ref[...], w_ref[...], preferred_element_type=jnp.float32)
    zs = z * dv
    dinv_ref[...] = dv
    zs_ref[0] = zs[:, :64]
    zs_ref[1] = zs[:, 64:]


def _tc_prep(xp, W0, deg01):
    return pl.pallas_call(
        _prep_kernel,
        grid=(8,),
        in_specs=[
            pl.BlockSpec((RB, 128), lambda i: (i, 0)),
            pl.BlockSpec((128, 128), lambda i: (0, 0)),
            pl.BlockSpec((2, RB, 16), lambda i: (0, i, 0)),
        ],
        out_specs=[
            pl.BlockSpec((2, RB, 64), lambda i: (0, i, 0)),
            pl.BlockSpec((RB, 1), lambda i: (i, 0)),
        ],
        out_shape=[
            jax.ShapeDtypeStruct((2, NP, 64), jnp.float32),
            jax.ShapeDtypeStruct((NP, 1), jnp.float32),
        ],
    )(xp, W0, deg01)


def _layer_kernel_res(acc_ref, zsp_ref, dinv_ref, b_ref, h_ref, w_ref,
                      hn_ref, zs_ref):
    dv = dinv_ref[...]
    m = jnp.concatenate([acc_ref[0] + zsp_ref[0], acc_ref[1] + zsp_ref[1]],
                        axis=1)
    h = dv * m + b_ref[...] + h_ref[...]
    hn_ref[...] = h
    z = jnp.dot(h, w_ref[...], preferred_element_type=jnp.float32)
    zs = z * dv
    zs_ref[0] = zs[:, :64]
    zs_ref[1] = zs[:, 64:]


def _layer_kernel_first(acc_ref, zsp_ref, dinv_ref, b_ref, w_ref,
                        hn_ref, zs_ref):
    dv = dinv_ref[...]
    m = jnp.concatenate([acc_ref[0] + zsp_ref[0], acc_ref[1] + zsp_ref[1]],
                        axis=1)
    h = dv * m + b_ref[...]
    hn_ref[...] = h
    z = jnp.dot(h, w_ref[...], preferred_element_type=jnp.float32)
    zs = z * dv
    zs_ref[0] = zs[:, :64]
    zs_ref[1] = zs[:, 64:]


def _tc_layer(acc, zsp, dinv, b, Wn, h_prev):
    if h_prev is None:
        kern = _layer_kernel_first
        args = (acc, zsp, dinv, b.reshape(1, 128), Wn)
        in_specs = [
            pl.BlockSpec((2, RB, 64), lambda i: (0, i, 0)),
            pl.BlockSpec((2, RB, 64), lambda i: (0, i, 0)),
            pl.BlockSpec((RB, 1), lambda i: (i, 0)),
            pl.BlockSpec((1, 128), lambda i: (0, 0)),
            pl.BlockSpec((128, 128), lambda i: (0, 0)),
        ]
    else:
        kern = _layer_kernel_res
        args = (acc, zsp, dinv, b.reshape(1, 128), h_prev, Wn)
        in_specs = [
            pl.BlockSpec((2, RB, 64), lambda i: (0, i, 0)),
            pl.BlockSpec((2, RB, 64), lambda i: (0, i, 0)),
            pl.BlockSpec((RB, 1), lambda i: (i, 0)),
            pl.BlockSpec((1, 128), lambda i: (0, 0)),
            pl.BlockSpec((RB, 128), lambda i: (i, 0)),
            pl.BlockSpec((128, 128), lambda i: (0, 0)),
        ]
    return pl.pallas_call(
        kern,
        grid=(8,),
        in_specs=in_specs,
        out_specs=[
            pl.BlockSpec((RB, 128), lambda i: (i, 0)),
            pl.BlockSpec((2, RB, 64), lambda i: (0, i, 0)),
        ],
        out_shape=[
            jax.ShapeDtypeStruct((NP, 128), jnp.float32),
            jax.ShapeDtypeStruct((2, NP, 64), jnp.float32),
        ],
    )(*args)


def _final_node_kernel(acc_ref, zsp_ref, dinv_ref, b_ref, h_ref, out_ref):
    dv = dinv_ref[...]
    m = jnp.concatenate([acc_ref[0] + zsp_ref[0], acc_ref[1] + zsp_ref[1]],
                        axis=1)
    h = dv * m + b_ref[...] + h_ref[...]
    out_ref[0] = h[:, :64]
    out_ref[1] = h[:, 64:]


def _tc_final_nodes(acc, zsp, dinv, b, h_prev):
    return pl.pallas_call(
        _final_node_kernel,
        grid=(8,),
        in_specs=[
            pl.BlockSpec((2, RB, 64), lambda i: (0, i, 0)),
            pl.BlockSpec((2, RB, 64), lambda i: (0, i, 0)),
            pl.BlockSpec((RB, 1), lambda i: (i, 0)),
            pl.BlockSpec((1, 128), lambda i: (0, 0)),
            pl.BlockSpec((RB, 128), lambda i: (i, 0)),
        ],
        out_specs=pl.BlockSpec((2, RB, 64), lambda i: (0, i, 0)),
        out_shape=jax.ShapeDtypeStruct((2, NP, 64), jnp.float32),
    )(acc, zsp, dinv, b.reshape(1, 128), h_prev)


def _readout_kernel(pool_ref, cnt_ref, idx_ref, wm_ref, bm_ref, out_ref):
    pool = jnp.concatenate(
        [pool_ref[0, :N_GRAPHS, :], pool_ref[1, :N_GRAPHS, :]], axis=1)
    cnt = cnt_ref[0, :N_GRAPHS, 0:1] + cnt_ref[1, :N_GRAPHS, 0:1]
    mean = pool / jnp.maximum(cnt, 1.0)
    gids = lax.broadcasted_iota(jnp.int32, (N_IDX, N_GRAPHS), 1)
    sel = (gids == idx_ref[...]).astype(jnp.float32)
    m = jnp.dot(sel, mean, preferred_element_type=jnp.float32)
    out_ref[...] = jnp.dot(m, wm_ref[...],
                           preferred_element_type=jnp.float32) + bm_ref[...]


def _tc_readout(pool01, cnt01, idx2, Wm, bm):
    return pl.pallas_call(
        _readout_kernel,
        out_shape=jax.ShapeDtypeStruct((N_IDX, D_OUT), jnp.float32),
    )(pool01, cnt01, idx2, Wm, bm.reshape(1, D_OUT))


_counts = _make_counts_kernel()
_edge_pass = _make_edge_kernel()
_pool_pass = _make_pool_kernel()


@jax.jit
def _run(x, edge_index, idx, batch, W0, b0, W1, b1, W2, b2, Wm, bm):
    ei32 = edge_index.astype(jnp.int32).reshape(2, NBLK, 1, 128)
    srcb = jnp.pad(ei32[0], ((0, SLAB), (0, 0), (0, 0)))
    dstb = jnp.pad(ei32[1], ((0, SLAB), (0, 0), (0, 0)))
    batchp = jnp.concatenate(
        [batch.astype(jnp.int32),
         jnp.full((NP - N_NODES,), N_GRAPHS, dtype=jnp.int32)]
    ).reshape(BBLK, 1, 128)
    xp = jnp.pad(x, ((0, NP - N_NODES), (0, 0)))
    idx2 = idx.astype(jnp.int32).reshape(N_IDX, 1)

    ones1 = jnp.ones((128, 16), jnp.float32)
    zeros1 = jnp.zeros((ROWS_PER_TILE, 16), jnp.float32)
    zeros64 = jnp.zeros((ROWS_PER_TILE, 64), jnp.float32)

    deg01, cnt01 = _counts(dstb, batchp, ones1, zeros1)
    zs0, dinv = _tc_prep(xp, W0, deg01)
    acc0 = _edge_pass(srcb, dstb, zs0, zeros64)
    h1, zs1 = _tc_layer(acc0, zs0, dinv, b0, W1, None)
    acc1 = _edge_pass(srcb, dstb, zs1, zeros64)
    h2, zs2 = _tc_layer(acc1, zs1, dinv, b1, W2, h1)
    acc2 = _edge_pass(srcb, dstb, zs2, zeros64)
    h3s = _tc_final_nodes(acc2, zs2, dinv, b2, h2)
    pool01 = _pool_pass(batchp, h3s, zeros64)
    return _tc_readout(pool01, cnt01, idx2, Wm, bm)


def kernel(x, edge_index, idx, batch, W0, b0, W1, b1, W2, b2, Wm, bm):
    return _run(x, edge_index, idx, batch, W0, b0, W1, b1, W2, b2, Wm, bm)


# speedup vs baseline: 1.0284x; 1.0284x over previous
import functools

import jax
import jax.numpy as jnp
from jax import lax
from jax.experimental import pallas as pl
from jax.experimental.pallas import tpu as pltpu
from jax.experimental.pallas import tpu_sc as plsc

N_NODES = 10000
N_EDGES = 320000
D_IN = 128
D_HID = 128
D_OUT = 64
N_GRAPHS = 512
N_IDX = 256

NP = 10112
NBLK = N_EDGES // 128
NSEX = 2496 // 6
SEXT = NSEX // 16
SLAB = 6 * SEXT + 4
BBLK = NP // 128
GP = N_GRAPHS + 8
ROWS_PER_TILE = NP // 16
CSLAB = NBLK // 32 + 1

_mesh = functools.partial(
    plsc.VectorSubcoreMesh, core_axis_name="c", subcore_axis_name="s")
_sc_params = pltpu.CompilerParams(use_tc_tiling_on_sc=False)


def _span(total, parts, p):
    base = total // parts
    rem = total % parts
    lo = p * base + jnp.minimum(p, rem)
    hi = lo + base + jnp.where(p < rem, 1, 0)
    return lo, hi


def _make_counts_kernel():
    @functools.partial(
        pl.kernel,
        out_type=(
            jax.ShapeDtypeStruct((2, NP, 16), jnp.float32),
            jax.ShapeDtypeStruct((2, GP, 16), jnp.float32),
        ),
        mesh=_mesh(),
        compiler_params=_sc_params,
        scratch_types=[
            pltpu.VMEM_SHARED((NP, 16), jnp.float32),
            pltpu.VMEM_SHARED((GP, 16), jnp.float32),
            pltpu.VMEM((CSLAB, 2, 128), jnp.int32),
            pltpu.VMEM((1, 128), jnp.int32),
            pltpu.VMEM((128, 16), jnp.float32),
            [pltpu.SemaphoreType.DMA] * 4,
        ],
    )
    def counts(edges_hbm, batch_hbm, ones_hbm, zeros1_hbm,
               deg_out, cnt_out, deg_sh, cnt_sh, islab, idx_v, ones_v, sems):
        c = lax.axis_index("c")
        s = lax.axis_index("s")
        w = c * 16 + s

        blo, bhi = _span(NBLK, 32, w)
        nb = bhi - blo
        pltpu.sync_copy(edges_hbm.at[pl.ds(blo, CSLAB)], islab)
        pltpu.sync_copy(ones_hbm, ones_v)
        pltpu.sync_copy(zeros1_hbm.at[pl.ds(0, ROWS_PER_TILE)],
                        deg_sh.at[pl.ds(s * ROWS_PER_TILE, ROWS_PER_TILE)])

        @pl.when(s == 0)
        def _zc():
            pltpu.sync_copy(zeros1_hbm.at[pl.ds(0, GP)], cnt_sh)

        plsc.subcore_barrier()

        def fire(i, k):
            pltpu.async_copy(ones_v, deg_sh.at[islab.at[i].at[1]], sems[k],
                             add=True)

        def drain(k):
            pltpu.make_async_copy(ones_v, deg_sh.at[islab.at[0].at[1]],
                                  sems[k]).wait()

        for k in range(4):
            fire(k, k)

        def body4(q, carry):
            i0 = 4 + q * 4
            for k in range(4):
                i = i0 + k

                @pl.when(i < nb)
                def _w(i=i, k=k):
                    drain(k)
                    fire(i, k)
            return carry

        lax.fori_loop(0, (nb + 3) // 4, body4, 0)

        for k in range(4):
            drain(k)

        lo2, hi2 = _span(BBLK, 32, w)

        def cnt_body(j, carry):
            pltpu.sync_copy(batch_hbm.at[j], idx_v)
            pltpu.sync_copy(ones_v, cnt_sh.at[idx_v.at[0]], add=True)
            return carry

        lax.fori_loop(lo2, hi2, cnt_body, 0)

        plsc.subcore_barrier()

        pltpu.sync_copy(deg_sh.at[pl.ds(s * ROWS_PER_TILE, ROWS_PER_TILE)],
                        deg_out.at[c].at[pl.ds(s * ROWS_PER_TILE,
                                               ROWS_PER_TILE)])

        @pl.when(s == 0)
        def _co():
            pltpu.sync_copy(cnt_sh, cnt_out.at[c])

    return counts


def _make_edge_kernel():
    @functools.partial(
        pl.kernel,
        out_type=jax.ShapeDtypeStruct((2, NP, 64), jnp.float32),
        mesh=_mesh(),
        compiler_params=_sc_params,
        scratch_types=[
            pltpu.VMEM_SHARED((NP, 64), jnp.float32),
            pltpu.VMEM((SLAB, 2, 128), jnp.int32),
            pltpu.VMEM((6, 128, 64), jnp.float32),
            pltpu.SemaphoreType.DMA,
            [pltpu.SemaphoreType.DMA] * 6,
            [pltpu.SemaphoreType.DMA] * 6,
        ],
    )
    def edge_pass(edges_hbm, table_hbm, zeros_hbm, acc_out,
                  acc_sh, islab, rows, isem, gsems, ssems):
        c = lax.axis_index("c")
        s = lax.axis_index("s")
        tbl = table_hbm.at[c]

        olo = s * SEXT
        ohi = olo + SEXT

        pltpu.async_copy(edges_hbm.at[pl.ds(6 * olo, SLAB)], islab, isem)
        pltpu.sync_copy(zeros_hbm,
                        acc_sh.at[pl.ds(s * ROWS_PER_TILE, ROWS_PER_TILE)])
        pltpu.make_async_copy(edges_hbm.at[pl.ds(6 * olo, SLAB)], islab,
                              isem).wait()
        plsc.subcore_barrier()

        def gather(l, k):
            pltpu.async_copy(tbl.at[islab.at[l].at[0]], rows.at[k], gsems[k])

        def gather_wait(k):
            pltpu.make_async_copy(tbl.at[islab.at[0].at[0]], rows.at[k],
                                  gsems[k]).wait()

        def scat(l, k):
            pltpu.async_copy(rows.at[k], acc_sh.at[islab.at[l].at[1]],
                             ssems[k], add=True)

        def scat_wait(k):
            pltpu.make_async_copy(rows.at[k], acc_sh.at[islab.at[0].at[1]],
                                  ssems[k]).wait()

        for k in range(6):
            gather(k, k)

        def body(o, carry):
            l0 = 6 * (o - olo)
            for k in range(6):
                gather_wait(k)
                scat(l0 + k, k)
            for k in range(6):
                scat_wait(k)

                @pl.when(o + 1 < ohi)
                def _pf(k=k, l0=l0):
                    gather(l0 + 6 + k, k)
            return carry

        lax.fori_loop(olo, ohi, body, 0)

        @pl.when(s == 15)
        def _tail():
            for k in range(4):
                gather(156 + k, k)
            for k in range(4):
                gather_wait(k)
                scat(156 + k, k)
            for k in range(4):
                scat_wait(k)

        plsc.subcore_barrier()

        pltpu.sync_copy(acc_sh.at[pl.ds(s * ROWS_PER_TILE, ROWS_PER_TILE)],
                        acc_out.at[c].at[pl.ds(s * ROWS_PER_TILE,
                                               ROWS_PER_TILE)])

    return edge_pass


def _make_pool_kernel():
    @functools.partial(
        pl.kernel,
        out_type=jax.ShapeDtypeStruct((2, GP, 64), jnp.float32),
        mesh=_mesh(),
        compiler_params=_sc_params,
        scratch_types=[
            pltpu.VMEM_SHARED((GP, 64), jnp.float32),
            pltpu.VMEM((1, 128), jnp.int32),
            pltpu.VMEM((128, 64), jnp.float32),
        ],
    )
    def pool_pass(batch_hbm, table_hbm, zeros_hbm,
                  pool_out, acc_sh, dstv, rows_v):
        c = lax.axis_index("c")
        s = lax.axis_index("s")

        @pl.when(s == 0)
        def _z():
            pltpu.sync_copy(zeros_hbm.at[pl.ds(0, GP)], acc_sh)

        plsc.subcore_barrier()

        lo, hi = _span(BBLK, 16, s)

        def body(j, carry):
            pltpu.sync_copy(batch_hbm.at[j], dstv)
            pltpu.sync_copy(table_hbm.at[c].at[pl.ds(j * 128, 128)], rows_v)
            pltpu.sync_copy(rows_v, acc_sh.at[dstv.at[0]], add=True)
            return carry

        lax.fori_loop(lo, hi, body, 0)

        plsc.subcore_barrier()

        @pl.when(s == 0)
        def _co():
            pltpu.sync_copy(acc_sh, pool_out.at[c])

    return pool_pass


RB = NP // 8


def _prep_kernel(x_ref, w_ref, deg_ref, zs_ref, dinv_ref):
    deg = deg_ref[0][:, 0:1] + deg_ref[1][:, 0:1] + 1.0
    dv = lax.rsqrt(deg)
    z = jnp.dot(x_ref[...], w_ref[...], preferred_element_type=jnp.float32)
    zs = z * dv
    dinv_ref[...] = dv
    zs_ref[0] = zs[:, :64]
    zs_ref[1] = zs[:, 64:]


def _tc_prep(xp, W0, deg01):
    return pl.pallas_call(
        _prep_kernel,
        grid=(8,),
        in_specs=[
            pl.BlockSpec((RB, 128), lambda i: (i, 0)),
            pl.BlockSpec((128, 128), lambda i: (0, 0)),
            pl.BlockSpec((2, RB, 16), lambda i: (0, i, 0)),
        ],
        out_specs=[
            pl.BlockSpec((2, RB, 64), lambda i: (0, i, 0)),
            pl.BlockSpec((RB, 1), lambda i: (i, 0)),
        ],
        out_shape=[
            jax.ShapeDtypeStruct((2, NP, 64), jnp.float32),
            jax.ShapeDtypeStruct((NP, 1), jnp.float32),
        ],
    )(xp, W0, deg01)


def _layer_kernel_res(acc_ref, zsp_ref, dinv_ref, b_ref, h_ref, w_ref,
                      hn_ref, zs_ref):
    dv = dinv_ref[...]
    m = jnp.concatenate([acc_ref[0] + zsp_ref[0], acc_ref[1] + zsp_ref[1]],
                        axis=1)
    h = dv * m + b_ref[...] + h_ref[...]
    hn_ref[...] = h
    z = jnp.dot(h, w_ref[...], preferred_element_type=jnp.float32)
    zs = z * dv
    zs_ref[0] = zs[:, :64]
    zs_ref[1] = zs[:, 64:]


def _layer_kernel_first(acc_ref, zsp_ref, dinv_ref, b_ref, w_ref,
                        hn_ref, zs_ref):
    dv = dinv_ref[...]
    m = jnp.concatenate([acc_ref[0] + zsp_ref[0], acc_ref[1] + zsp_ref[1]],
                        axis=1)
    h = dv * m + b_ref[...]
    hn_ref[...] = h
    z = jnp.dot(h, w_ref[...], preferred_element_type=jnp.float32)
    zs = z * dv
    zs_ref[0] = zs[:, :64]
    zs_ref[1] = zs[:, 64:]


def _tc_layer(acc, zsp, dinv, b, Wn, h_prev):
    if h_prev is None:
        kern = _layer_kernel_first
        args = (acc, zsp, dinv, b.reshape(1, 128), Wn)
        in_specs = [
            pl.BlockSpec((2, RB, 64), lambda i: (0, i, 0)),
            pl.BlockSpec((2, RB, 64), lambda i: (0, i, 0)),
            pl.BlockSpec((RB, 1), lambda i: (i, 0)),
            pl.BlockSpec((1, 128), lambda i: (0, 0)),
            pl.BlockSpec((128, 128), lambda i: (0, 0)),
        ]
    else:
        kern = _layer_kernel_res
        args = (acc, zsp, dinv, b.reshape(1, 128), h_prev, Wn)
        in_specs = [
            pl.BlockSpec((2, RB, 64), lambda i: (0, i, 0)),
            pl.BlockSpec((2, RB, 64), lambda i: (0, i, 0)),
            pl.BlockSpec((RB, 1), lambda i: (i, 0)),
            pl.BlockSpec((1, 128), lambda i: (0, 0)),
            pl.BlockSpec((RB, 128), lambda i: (i, 0)),
            pl.BlockSpec((128, 128), lambda i: (0, 0)),
        ]
    return pl.pallas_call(
        kern,
        grid=(8,),
        in_specs=in_specs,
        out_specs=[
            pl.BlockSpec((RB, 128), lambda i: (i, 0)),
            pl.BlockSpec((2, RB, 64), lambda i: (0, i, 0)),
        ],
        out_shape=[
            jax.ShapeDtypeStruct((NP, 128), jnp.float32),
            jax.ShapeDtypeStruct((2, NP, 64), jnp.float32),
        ],
    )(*args)


def _final_node_kernel(acc_ref, zsp_ref, dinv_ref, b_ref, h_ref, out_ref):
    dv = dinv_ref[...]
    m = jnp.concatenate([acc_ref[0] + zsp_ref[0], acc_ref[1] + zsp_ref[1]],
                        axis=1)
    h = dv * m + b_ref[...] + h_ref[...]
    out_ref[0] = h[:, :64]
    out_ref[1] = h[:, 64:]


def _tc_final_nodes(acc, zsp, dinv, b, h_prev):
    return pl.pallas_call(
        _final_node_kernel,
        grid=(8,),
        in_specs=[
            pl.BlockSpec((2, RB, 64), lambda i: (0, i, 0)),
            pl.BlockSpec((2, RB, 64), lambda i: (0, i, 0)),
            pl.BlockSpec((RB, 1), lambda i: (i, 0)),
            pl.BlockSpec((1, 128), lambda i: (0, 0)),
            pl.BlockSpec((RB, 128), lambda i: (i, 0)),
        ],
        out_specs=pl.BlockSpec((2, RB, 64), lambda i: (0, i, 0)),
        out_shape=jax.ShapeDtypeStruct((2, NP, 64), jnp.float32),
    )(acc, zsp, dinv, b.reshape(1, 128), h_prev)


def _readout_kernel(pool_ref, cnt_ref, idx_ref, wm_ref, bm_ref, out_ref):
    pool = jnp.concatenate(
        [pool_ref[0, :N_GRAPHS, :], pool_ref[1, :N_GRAPHS, :]], axis=1)
    cnt = cnt_ref[0, :N_GRAPHS, 0:1] + cnt_ref[1, :N_GRAPHS, 0:1]
    mean = pool / jnp.maximum(cnt, 1.0)
    gids = lax.broadcasted_iota(jnp.int32, (N_IDX, N_GRAPHS), 1)
    sel = (gids == idx_ref[...]).astype(jnp.float32)
    m = jnp.dot(sel, mean, preferred_element_type=jnp.float32)
    out_ref[...] = jnp.dot(m, wm_ref[...],
                           preferred_element_type=jnp.float32) + bm_ref[...]


def _tc_readout(pool01, cnt01, idx2, Wm, bm):
    return pl.pallas_call(
        _readout_kernel,
        out_shape=jax.ShapeDtypeStruct((N_IDX, D_OUT), jnp.float32),
    )(pool01, cnt01, idx2, Wm, bm.reshape(1, D_OUT))


_counts = _make_counts_kernel()
_edge_pass = _make_edge_kernel()
_pool_pass = _make_pool_kernel()


@jax.jit
def _run(x, edge_index, idx, batch, W0, b0, W1, b1, W2, b2, Wm, bm):
    ei32 = edge_index.astype(jnp.int32).reshape(2, NBLK, 128)
    edges = jnp.stack([ei32[0], ei32[1]], axis=1)
    edges = jnp.pad(edges, ((0, SLAB), (0, 0), (0, 0)))
    batchp = jnp.concatenate(
        [batch.astype(jnp.int32),
         jnp.full((NP - N_NODES,), N_GRAPHS, dtype=jnp.int32)]
    ).reshape(BBLK, 1, 128)
    xp = jnp.pad(x, ((0, NP - N_NODES), (0, 0)))
    idx2 = idx.astype(jnp.int32).reshape(N_IDX, 1)

    ones1 = jnp.ones((128, 16), jnp.float32)
    zeros1 = jnp.zeros((ROWS_PER_TILE, 16), jnp.float32)
    zeros64 = jnp.zeros((ROWS_PER_TILE, 64), jnp.float32)

    deg01, cnt01 = _counts(edges, batchp, ones1, zeros1)
    zs0, dinv = _tc_prep(xp, W0, deg01)
    acc0 = _edge_pass(edges, zs0, zeros64)
    h1, zs1 = _tc_layer(acc0, zs0, dinv, b0, W1, None)
    acc1 = _edge_pass(edges, zs1, zeros64)
    h2, zs2 = _tc_layer(acc1, zs1, dinv, b1, W2, h1)
    acc2 = _edge_pass(edges, zs2, zeros64)
    h3s = _tc_final_nodes(acc2, zs2, dinv, b2, h2)
    pool01 = _pool_pass(batchp, h3s, zeros64)
    return _tc_readout(pool01, cnt01, idx2, Wm, bm)


def kernel(x, edge_index, idx, batch, W0, b0, W1, b1, W2, b2, Wm, bm):
    return _run(x, edge_index, idx, batch, W0, b0, W1, b1, W2, b2, Wm, bm)
